# baseline (speedup 1.0000x reference)
import jax
import jax.numpy as jnp
from jax.experimental import pallas as pl
from jax.experimental.pallas import tpu as pltpu

_N, _P, _D = 4, 4096, 3
_QC = 2048
_NQ = _P // _QC


def _chamfer_kernel(src_ref, tgt_ref, out_ref):
    def step(i, carry):
        m_run, loss = carry
        b = i // _NQ
        j = i % _NQ

        St = src_ref[b]
        T = tgt_ref[b, pl.ds(j * _QC, _QC), :]

        x2 = jnp.sum(St * St, axis=0, keepdims=True)
        y2 = jnp.sum(T * T, axis=1, keepdims=True)

        x2_hi = x2.astype(jnp.bfloat16).astype(jnp.float32)
        x2_lo = x2 - x2_hi
        y2_hi = y2.astype(jnp.bfloat16).astype(jnp.float32)
        y2_lo = y2 - y2_hi
        ones_p = jnp.ones((1, _P), jnp.float32)
        ones_q = jnp.ones((_QC, 1), jnp.float32)
        L = jnp.concatenate([T, y2_hi, y2_lo, ones_q, ones_q],
                            axis=1)
        R = jnp.concatenate([-2.0 * St, ones_p, ones_p, x2_hi, x2_lo],
                            axis=0)

        d = jax.lax.dot_general(
            L, R, (((1,), (0,)), ((), ())),
            preferred_element_type=jnp.float32,
        )
        m = jnp.min(d, axis=0, keepdims=True)

        m_run = jnp.where(j == 0, m, jnp.minimum(m_run, m))
        loss = jnp.where(j == _NQ - 1,
                         loss + jnp.sum(m_run, keepdims=True),
                         loss)
        return m_run, loss

    init = (jnp.full((1, _P), jnp.inf, jnp.float32),
            jnp.zeros((1, 1), jnp.float32))
    _, loss = jax.lax.fori_loop(0, _N * _NQ, step, init)
    out_ref[...] = loss * (1.0 / _N)


def kernel(source_cloud, target_cloud):
    src_t = source_cloud.transpose(0, 2, 1)
    out = pl.pallas_call(
        _chamfer_kernel,
        grid=(1,),
        in_specs=[
            pl.BlockSpec((_N, _D, _P), lambda i: (0, 0, 0)),
            pl.BlockSpec((_N, _P, _D), lambda i: (0, 0, 0)),
        ],
        out_specs=pl.BlockSpec((1, 1), lambda i: (0, 0)),
        out_shape=jax.ShapeDtypeStruct((1, 1), jnp.float32),
    )(src_t, target_cloud)
    return out[0, 0]


# speedup vs baseline: 1.0695x; 1.0695x over previous
import jax
import jax.numpy as jnp
from jax.experimental import pallas as pl
from jax.experimental.pallas import tpu as pltpu

_N, _P, _D = 4, 4096, 3
_QC = 2048
_NQ = _P // _QC


def _chamfer_kernel(src_ref, tgt_ref, out_ref):
    b = pl.program_id(0)

    St = src_ref[0]
    T = tgt_ref[0]

    x2 = jnp.sum(St * St, axis=0, keepdims=True)
    y2 = jnp.sum(T * T, axis=1, keepdims=True)

    x2_hi = x2.astype(jnp.bfloat16).astype(jnp.float32)
    x2_lo = x2 - x2_hi
    y2_hi = y2.astype(jnp.bfloat16).astype(jnp.float32)
    y2_lo = y2 - y2_hi
    ones_p = jnp.ones((1, _P), jnp.float32)
    ones_q = jnp.ones((_P, 1), jnp.float32)
    L = jnp.concatenate([T, y2_hi, y2_lo, ones_q, ones_q],
                        axis=1)
    R = jnp.concatenate([-2.0 * St, ones_p, ones_p, x2_hi, x2_lo],
                        axis=0)

    m = None
    for j in range(_NQ):
        d = jax.lax.dot_general(
            L[j * _QC:(j + 1) * _QC], R, (((1,), (0,)), ((), ())),
            preferred_element_type=jnp.float32,
        )
        mj = jnp.min(d, axis=0, keepdims=True)
        m = mj if m is None else jnp.minimum(m, mj)

    s = jnp.sum(m, keepdims=True) * (1.0 / _N)

    @pl.when(b == 0)
    def _():
        out_ref[...] = jnp.zeros_like(out_ref)

    out_ref[...] += s


def kernel(source_cloud, target_cloud):
    src_t = source_cloud.transpose(0, 2, 1)
    out = pl.pallas_call(
        _chamfer_kernel,
        grid=(_N,),
        in_specs=[
            pl.BlockSpec((1, _D, _P), lambda b: (b, 0, 0)),
            pl.BlockSpec((1, _P, _D), lambda b: (b, 0, 0)),
        ],
        out_specs=pl.BlockSpec((1, 1), lambda b: (0, 0)),
        out_shape=jax.ShapeDtypeStruct((1, 1), jnp.float32),
    )(src_t, target_cloud)
    return out[0, 0]
